# baseline (speedup 1.0000x reference)
import functools

import jax
import jax.numpy as jnp
from jax import lax
from jax.experimental import pallas as pl
from jax.experimental.pallas import tpu as pltpu
from jax.experimental.pallas import tpu_sc as plsc

_BN = 512


def _score_body(adj_ref, xf_ref, w_ref, sig_ref):
    adj = adj_ref[0]
    xf = xf_ref[0]
    out = jnp.dot(adj, xf, preferred_element_type=jnp.float32)
    s = jnp.dot(out, w_ref[...], preferred_element_type=jnp.float32)
    sig_ref[...] = (1.0 / (1.0 + jnp.exp(-s)))[None]


def _scores(adj, xf, W):
    B, N, K = xf.shape
    return pl.pallas_call(
        _score_body,
        grid=(B, N // _BN),
        in_specs=[
            pl.BlockSpec((1, _BN, N), lambda b, nb: (b, nb, 0)),
            pl.BlockSpec((1, N, K), lambda b, nb: (b, 0, 0)),
            pl.BlockSpec((K, 1), lambda b, nb: (0, 0)),
        ],
        out_specs=pl.BlockSpec((1, _BN, 1), lambda b, nb: (b, nb, 0)),
        out_shape=jax.ShapeDtypeStruct((B, N, 1), jnp.float32),
    )(adj, xf, W)


def _sort_body(sig_ref, top_ref):
    N = sig_ref.shape[2]
    L = 128
    M = N // L
    key = lax.bitcast_convert_type(sig_ref[...], jnp.int32).reshape(M, L)
    lane = lax.broadcasted_iota(jnp.int32, (M, L), 1)
    row = lax.broadcasted_iota(jnp.int32, (M, L), 0)
    idx = row * L + lane
    k = 2
    while k <= N:
        j = k // 2
        while j >= 1:
            if j < L:
                lower = (lane & j) == 0
                ax, d = 1, j
            else:
                d = j // L
                lower = (row & d) == 0
                ax = 0
            kp = jnp.where(lower, jnp.roll(key, -d, axis=ax), jnp.roll(key, d, axis=ax))
            ip = jnp.where(lower, jnp.roll(idx, -d, axis=ax), jnp.roll(idx, d, axis=ax))
            asc = ((lane & k) == 0) if k < L else ((row & (k // L)) == 0)
            klo = jnp.where(lower, key, kp)
            khi = jnp.where(lower, kp, key)
            ilo = jnp.where(lower, idx, ip)
            ihi = jnp.where(lower, ip, idx)
            good = (klo > khi) | ((klo == khi) & (ilo < ihi))
            swap = good ^ asc
            key = jnp.where(swap, kp, key)
            idx = jnp.where(swap, ip, idx)
            j //= 2
        k *= 2
    b = pl.program_id(0)
    top_ref[...] = (idx[: M // 2, :] + b * N).reshape(1, 1, N // 2)


def _sort(sig):
    B, N = sig.shape
    out = pl.pallas_call(
        _sort_body,
        grid=(B,),
        in_specs=[pl.BlockSpec((1, 1, N), lambda b: (b, 0, 0))],
        out_specs=pl.BlockSpec((1, 1, N // 2), lambda b: (b, 0, 0)),
        out_shape=jax.ShapeDtypeStruct((B, 1, N // 2), jnp.int32),
    )(sig.reshape(B, 1, N))
    return out.reshape(B, N // 2)


def _gather_body(xf_hbm, idx_hbm, out_hbm, idx_v, rows_v, sem):
    wid = lax.axis_index("s") * 2 + lax.axis_index("c")
    pltpu.sync_copy(idx_hbm.at[pl.ds(wid * 2, 2)], idx_v)
    for c in range(2):
        pltpu.async_copy(xf_hbm.at[idx_v.at[c]], rows_v.at[c], sem).wait()
    for c in range(2):
        pltpu.sync_copy(rows_v.at[c], out_hbm.at[pl.ds(wid * 256 + c * 128, 128)])


def _gather(xf_flat, idx_2d):
    R, K = xf_flat.shape
    nrows = idx_2d.shape[0] * idx_2d.shape[1]
    return pl.kernel(
        _gather_body,
        out_type=jax.ShapeDtypeStruct((nrows, K), jnp.float32),
        mesh=plsc.VectorSubcoreMesh(core_axis_name="c", subcore_axis_name="s"),
        scratch_types=[
            pltpu.VMEM((2, 128), jnp.int32),
            pltpu.VMEM((2, 128, K), jnp.float32),
            pltpu.SemaphoreType.DMA,
        ],
    )(xf_flat, idx_2d)


@jax.jit
def kernel(x, adj, W):
    B, F, N, T = x.shape
    K = T * F
    xf = jnp.transpose(x, (0, 2, 3, 1)).reshape(B, N, K)
    sig = _scores(adj, xf, W)
    top = _sort(sig.reshape(B, N))
    xf_pad = jnp.pad(xf.reshape(B * N, K), ((0, 0), (0, 128 - K)))
    g = _gather(xf_pad, top.reshape(-1, 128))[:, :K]
    return g.reshape(B, N // 2, T, F).transpose(0, 3, 1, 2)


# speedup vs baseline: 1.1937x; 1.1937x over previous
import jax
import jax.numpy as jnp
from jax import lax
from jax.experimental import pallas as pl
from jax.experimental.pallas import tpu as pltpu
from jax.experimental.pallas import tpu_sc as plsc

_BN = 512
_NB = 8


def _sort_tile(sig_tile, b, N):
    L = 128
    M = N // L
    key = lax.bitcast_convert_type(sig_tile, jnp.int32)
    lane = lax.broadcasted_iota(jnp.int32, (M, L), 1)
    row = lax.broadcasted_iota(jnp.int32, (M, L), 0)
    idx = row * L + lane
    k = 2
    while k <= N:
        j = k // 2
        while j >= 1:
            if j < L:
                lower = (lane & j) == 0
                ax, d = 1, j
            else:
                d = j // L
                lower = (row & d) == 0
                ax = 0
            kp = jnp.where(lower, jnp.roll(key, -d, axis=ax), jnp.roll(key, d, axis=ax))
            ip = jnp.where(lower, jnp.roll(idx, -d, axis=ax), jnp.roll(idx, d, axis=ax))
            asc = ((lane & k) == 0) if k < L else ((row & (k // L)) == 0)
            klo = jnp.where(lower, key, kp)
            khi = jnp.where(lower, kp, key)
            ilo = jnp.where(lower, idx, ip)
            ihi = jnp.where(lower, ip, idx)
            good = (klo > khi) | ((klo == khi) & (ilo < ihi))
            swap = good ^ asc
            key = jnp.where(swap, kp, key)
            idx = jnp.where(swap, ip, idx)
            j //= 2
        k *= 2
    return (idx[: M // 2, :] + b * N).reshape(1, 1, N // 2)


def _fused_body(adj_ref, xf_ref, w_ref, top_ref, xfp_ref, sig_acc):
    nb = pl.program_id(1)
    b = pl.program_id(0)
    N = xf_ref.shape[1]

    @pl.when(nb == 0)
    def _emit_table():
        xf = xf_ref[0]
        xfp_ref[...] = jnp.concatenate(
            [xf, jnp.zeros((N, 128 - xf.shape[1]), jnp.float32)], axis=1
        )[None]

    @pl.when(nb < _NB)
    def _score_step():
        adj = adj_ref[0]
        out = jnp.dot(adj, xf_ref[0], preferred_element_type=jnp.float32)
        s = jnp.dot(out, w_ref[...], preferred_element_type=jnp.float32)
        sig = 1.0 / (1.0 + jnp.exp(-s))
        sig_acc[pl.ds(nb * (_BN // 128), _BN // 128), :] = sig.reshape(_BN // 128, 128)

    @pl.when(nb == _NB)
    def _sort_step():
        top_ref[...] = _sort_tile(sig_acc[...], b, N)


def _fused(adj, xf, W):
    B, N, K = xf.shape
    return pl.pallas_call(
        _fused_body,
        grid=(B, _NB + 1),
        in_specs=[
            pl.BlockSpec((1, _BN, N), lambda b, nb: (b, jnp.minimum(nb, _NB - 1), 0)),
            pl.BlockSpec((1, N, K), lambda b, nb: (b, 0, 0)),
            pl.BlockSpec((K, 1), lambda b, nb: (0, 0)),
        ],
        out_specs=[
            pl.BlockSpec((1, 1, N // 2), lambda b, nb: (b, 0, 0)),
            pl.BlockSpec((1, N, 128), lambda b, nb: (b, 0, 0)),
        ],
        out_shape=[
            jax.ShapeDtypeStruct((B, 1, N // 2), jnp.int32),
            jax.ShapeDtypeStruct((B, N, 128), jnp.float32),
        ],
        scratch_shapes=[pltpu.VMEM((N // 128, 128), jnp.float32)],
    )(adj, xf, W)


def _gather_body(xf_hbm, idx_hbm, out_hbm, idx_v, rows_v, sem):
    wid = lax.axis_index("s") * 2 + lax.axis_index("c")
    pltpu.sync_copy(idx_hbm.at[pl.ds(wid * 2, 2)], idx_v)
    for c in range(2):
        pltpu.async_copy(xf_hbm.at[idx_v.at[c]], rows_v.at[c], sem).wait()
    for c in range(2):
        pltpu.sync_copy(rows_v.at[c], out_hbm.at[pl.ds(wid * 256 + c * 128, 128)])


def _gather(xf_flat, idx_2d):
    R, K = xf_flat.shape
    nrows = idx_2d.shape[0] * idx_2d.shape[1]
    return pl.kernel(
        _gather_body,
        out_type=jax.ShapeDtypeStruct((nrows, K), jnp.float32),
        mesh=plsc.VectorSubcoreMesh(core_axis_name="c", subcore_axis_name="s"),
        scratch_types=[
            pltpu.VMEM((2, 128), jnp.int32),
            pltpu.VMEM((2, 128, K), jnp.float32),
            pltpu.SemaphoreType.DMA,
        ],
    )(xf_flat, idx_2d)


@jax.jit
def kernel(x, adj, W):
    B, F, N, T = x.shape
    K = T * F
    xf = jnp.transpose(x, (0, 2, 3, 1)).reshape(B, N, K)
    top, xf_pad = _fused(adj, xf, W)
    g = _gather(xf_pad.reshape(B * N, 128), top.reshape(-1, 128))[:, :K]
    return g.reshape(B, N // 2, T, F).transpose(0, 3, 1, 2)


# speedup vs baseline: 1.2090x; 1.0128x over previous
import jax
import jax.numpy as jnp
from jax import lax
from jax.experimental import pallas as pl
from jax.experimental.pallas import tpu as pltpu
from jax.experimental.pallas import tpu_sc as plsc

_BN = 1024
_NB = 4


def _sort_tile(sig_tile, b, N):
    L = 128
    M = N // L
    key = lax.bitcast_convert_type(sig_tile, jnp.int32)
    lane = lax.broadcasted_iota(jnp.int32, (M, L), 1)
    row = lax.broadcasted_iota(jnp.int32, (M, L), 0)
    idx = row * L + lane
    k = 2
    while k <= N:
        j = k // 2
        while j >= 1:
            if j < L:
                lower = (lane & j) == 0
                ax, d = 1, j
            else:
                d = j // L
                lower = (row & d) == 0
                ax = 0
            kp = jnp.where(lower, jnp.roll(key, -d, axis=ax), jnp.roll(key, d, axis=ax))
            ip = jnp.where(lower, jnp.roll(idx, -d, axis=ax), jnp.roll(idx, d, axis=ax))
            asc = ((lane & k) == 0) if k < L else ((row & (k // L)) == 0)
            klo = jnp.where(lower, key, kp)
            khi = jnp.where(lower, kp, key)
            ilo = jnp.where(lower, idx, ip)
            ihi = jnp.where(lower, ip, idx)
            good = (klo > khi) | ((klo == khi) & (ilo < ihi))
            swap = good ^ asc
            key = jnp.where(swap, kp, key)
            idx = jnp.where(swap, ip, idx)
            j //= 2
        k *= 2
    return (idx[: M // 2, :] + b * N).reshape(1, 1, N // 2)


def _fused_body(adj_ref, xf_ref, w_ref, top_ref, xfp_ref, sig_acc):
    nb = pl.program_id(1)
    b = pl.program_id(0)
    N = xf_ref.shape[1]

    @pl.when(nb == 0)
    def _emit_table():
        xf = xf_ref[0]
        xfp_ref[...] = jnp.concatenate(
            [xf, jnp.zeros((N, 128 - xf.shape[1]), jnp.float32)], axis=1
        )[None]

    @pl.when(nb < _NB)
    def _score_step():
        adj = adj_ref[0]
        out = jnp.dot(adj, xf_ref[0], preferred_element_type=jnp.float32)
        s = jnp.dot(out, w_ref[...], preferred_element_type=jnp.float32)
        sig = 1.0 / (1.0 + jnp.exp(-s))
        sig_acc[pl.ds(nb * (_BN // 128), _BN // 128), :] = sig.reshape(_BN // 128, 128)

    @pl.when(nb == _NB)
    def _sort_step():
        top_ref[...] = _sort_tile(sig_acc[...], b, N)


def _fused(adj, xf, W):
    B, N, K = xf.shape
    return pl.pallas_call(
        _fused_body,
        grid=(B, _NB + 1),
        in_specs=[
            pl.BlockSpec((1, _BN, N), lambda b, nb: (b, jnp.minimum(nb, _NB - 1), 0)),
            pl.BlockSpec((1, N, K), lambda b, nb: (b, 0, 0)),
            pl.BlockSpec((K, 1), lambda b, nb: (0, 0)),
        ],
        out_specs=[
            pl.BlockSpec((1, 1, N // 2), lambda b, nb: (b, 0, 0)),
            pl.BlockSpec((1, N, 128), lambda b, nb: (b, 0, 0)),
        ],
        out_shape=[
            jax.ShapeDtypeStruct((B, 1, N // 2), jnp.int32),
            jax.ShapeDtypeStruct((B, N, 128), jnp.float32),
        ],
        scratch_shapes=[pltpu.VMEM((N // 128, 128), jnp.float32)],
    )(adj, xf, W)


def _gather_body(xf_hbm, idx_hbm, out_hbm, idx_v, rows_v, sem):
    wid = lax.axis_index("s") * 2 + lax.axis_index("c")
    pltpu.sync_copy(idx_hbm.at[pl.ds(wid * 2, 2)], idx_v)
    for c in range(2):
        pltpu.async_copy(xf_hbm.at[idx_v.at[c]], rows_v.at[c], sem).wait()
    for c in range(2):
        pltpu.sync_copy(rows_v.at[c], out_hbm.at[pl.ds(wid * 256 + c * 128, 128)])


def _gather(xf_flat, idx_2d):
    R, K = xf_flat.shape
    nrows = idx_2d.shape[0] * idx_2d.shape[1]
    return pl.kernel(
        _gather_body,
        out_type=jax.ShapeDtypeStruct((nrows, K), jnp.float32),
        mesh=plsc.VectorSubcoreMesh(core_axis_name="c", subcore_axis_name="s"),
        scratch_types=[
            pltpu.VMEM((2, 128), jnp.int32),
            pltpu.VMEM((2, 128, K), jnp.float32),
            pltpu.SemaphoreType.DMA,
        ],
    )(xf_flat, idx_2d)


@jax.jit
def kernel(x, adj, W):
    B, F, N, T = x.shape
    K = T * F
    xf = jnp.transpose(x, (0, 2, 3, 1)).reshape(B, N, K)
    top, xf_pad = _fused(adj, xf, W)
    g = _gather(xf_pad.reshape(B * N, 128), top.reshape(-1, 128))[:, :K]
    return g.reshape(B, N // 2, T, F).transpose(0, 3, 1, 2)


# speedup vs baseline: 1.2985x; 1.0741x over previous
import jax
import jax.numpy as jnp
from jax import lax
from jax.experimental import pallas as pl
from jax.experimental.pallas import tpu as pltpu
from jax.experimental.pallas import tpu_sc as plsc

_BN = 1024
_NB = 4
_B = 4


def _sort_tiles(sig, B, N):
    L = 128
    M = N // L
    key = lax.bitcast_convert_type(sig, jnp.int32).reshape(B, M, L)
    lane = lax.broadcasted_iota(jnp.int32, (B, M, L), 2)
    row = lax.broadcasted_iota(jnp.int32, (B, M, L), 1)
    idx = row * L + lane
    k = 2
    while k <= N:
        j = k // 2
        while j >= 1:
            if j < L:
                lower = (lane & j) == 0
                ax, d = 2, j
            else:
                d = j // L
                lower = (row & d) == 0
                ax = 1
            kp = jnp.where(lower, jnp.roll(key, -d, axis=ax), jnp.roll(key, d, axis=ax))
            ip = jnp.where(lower, jnp.roll(idx, -d, axis=ax), jnp.roll(idx, d, axis=ax))
            asc = ((lane & k) == 0) if k < L else ((row & (k // L)) == 0)
            klo = jnp.where(lower, key, kp)
            khi = jnp.where(lower, kp, key)
            ilo = jnp.where(lower, idx, ip)
            ihi = jnp.where(lower, ip, idx)
            good = (klo > khi) | ((klo == khi) & (ilo < ihi))
            swap = good ^ asc
            key = jnp.where(swap, kp, key)
            idx = jnp.where(swap, ip, idx)
            j //= 2
        k *= 2
    base = lax.broadcasted_iota(jnp.int32, (B, M // 2, L), 0) * N
    return (idx[:, : M // 2, :] + base).reshape(B, 1, N // 2)


def _fused_body(adj_ref, xf_ref, w_ref, top_ref, xfp_ref, sig_acc):
    i = pl.program_id(0)
    nsteps = pl.num_programs(0)
    N = xf_ref.shape[1]
    rows = _BN // 128

    @pl.when(i % _NB == 0)
    def _emit_table():
        xf = xf_ref[0]
        xfp_ref[...] = jnp.concatenate(
            [xf, jnp.zeros((N, 128 - xf.shape[1]), jnp.float32)], axis=1
        )[None]

    @pl.when(i < nsteps - 1)
    def _score_step():
        adj = adj_ref[0]
        out = jnp.dot(adj, xf_ref[0], preferred_element_type=jnp.float32)
        s = jnp.dot(out, w_ref[...], preferred_element_type=jnp.float32)
        sig = 1.0 / (1.0 + jnp.exp(-s))
        sig_acc[pl.ds(i * rows, rows), :] = sig.reshape(rows, 128)

    @pl.when(i == nsteps - 1)
    def _sort_step():
        top_ref[...] = _sort_tiles(sig_acc[...], _B, N)


def _fused(adj, xf, W):
    B, N, K = xf.shape
    last = B * _NB - 1
    return pl.pallas_call(
        _fused_body,
        grid=(B * _NB + 1,),
        in_specs=[
            pl.BlockSpec(
                (1, _BN, N),
                lambda i: (jnp.minimum(i, last) // _NB, jnp.minimum(i, last) % _NB, 0),
            ),
            pl.BlockSpec((1, N, K), lambda i: (jnp.minimum(i // _NB, _B - 1), 0, 0)),
            pl.BlockSpec((K, 1), lambda i: (0, 0)),
        ],
        out_specs=[
            pl.BlockSpec((B, 1, N // 2), lambda i: (0, 0, 0)),
            pl.BlockSpec((1, N, 128), lambda i: (jnp.minimum(i // _NB, _B - 1), 0, 0)),
        ],
        out_shape=[
            jax.ShapeDtypeStruct((B, 1, N // 2), jnp.int32),
            jax.ShapeDtypeStruct((B, N, 128), jnp.float32),
        ],
        scratch_shapes=[pltpu.VMEM((B * N // 128, 128), jnp.float32)],
    )(adj, xf, W)


def _gather_body(xf_hbm, idx_hbm, out_hbm, idx_v, rows_v, sem):
    wid = lax.axis_index("s") * 2 + lax.axis_index("c")
    pltpu.sync_copy(idx_hbm.at[pl.ds(wid * 2, 2)], idx_v)
    cps = [
        pltpu.async_copy(xf_hbm.at[idx_v.at[c]], rows_v.at[c], sem) for c in range(2)
    ]
    for cp in cps:
        cp.wait()
    for c in range(2):
        pltpu.sync_copy(rows_v.at[c], out_hbm.at[pl.ds(wid * 256 + c * 128, 128)])


def _gather(xf_flat, idx_2d):
    R, K = xf_flat.shape
    nrows = idx_2d.shape[0] * idx_2d.shape[1]
    return pl.kernel(
        _gather_body,
        out_type=jax.ShapeDtypeStruct((nrows, K), jnp.float32),
        mesh=plsc.VectorSubcoreMesh(core_axis_name="c", subcore_axis_name="s"),
        scratch_types=[
            pltpu.VMEM((2, 128), jnp.int32),
            pltpu.VMEM((2, 128, K), jnp.float32),
            pltpu.SemaphoreType.DMA,
        ],
    )(xf_flat, idx_2d)


@jax.jit
def kernel(x, adj, W):
    B, F, N, T = x.shape
    K = T * F
    xf = jnp.transpose(x, (0, 2, 3, 1)).reshape(B, N, K)
    top, xf_pad = _fused(adj, xf, W)
    g = _gather(xf_pad.reshape(B * N, 128), top.reshape(-1, 128))[:, :K]
    return g.reshape(B, N // 2, T, F).transpose(0, 3, 1, 2)


# speedup vs baseline: 1.3088x; 1.0079x over previous
import jax
import jax.numpy as jnp
from jax import lax
from jax.experimental import pallas as pl
from jax.experimental.pallas import tpu as pltpu
from jax.experimental.pallas import tpu_sc as plsc

_BN = 512
_NB = 8


def _sort_tile(sig_tile, b, N):
    L = 128
    M = N // L
    key = lax.bitcast_convert_type(sig_tile, jnp.int32)
    lane = lax.broadcasted_iota(jnp.int32, (M, L), 1)
    row = lax.broadcasted_iota(jnp.int32, (M, L), 0)
    idx = row * L + lane
    k = 2
    while k <= N:
        j = k // 2
        while j >= 1:
            if j < L:
                lower = (lane & j) == 0
                ax, d = 1, j
            else:
                d = j // L
                lower = (row & d) == 0
                ax = 0
            kp = jnp.where(lower, jnp.roll(key, -d, axis=ax), jnp.roll(key, d, axis=ax))
            ip = jnp.where(lower, jnp.roll(idx, -d, axis=ax), jnp.roll(idx, d, axis=ax))
            asc = ((lane & k) == 0) if k < L else ((row & (k // L)) == 0)
            klo = jnp.where(lower, key, kp)
            khi = jnp.where(lower, kp, key)
            ilo = jnp.where(lower, idx, ip)
            ihi = jnp.where(lower, ip, idx)
            good = (klo > khi) | ((klo == khi) & (ilo < ihi))
            swap = good ^ asc
            key = jnp.where(swap, kp, key)
            idx = jnp.where(swap, ip, idx)
            j //= 2
        k *= 2
    return (idx[: M // 2, :] + b * N).reshape(1, 1, N // 2)


def _fused_body(adj_ref, xf_ref, w_ref, top_ref, xfp_ref, sig_acc):
    i = pl.program_id(0)
    nsteps = pl.num_programs(0)
    N = xf_ref.shape[1]
    rows = _BN // 128

    @pl.when((i % _NB == 0) & (i < nsteps - 1))
    def _emit_table():
        xf = xf_ref[0]
        xfp_ref[...] = jnp.concatenate(
            [xf, jnp.zeros((N, 128 - xf.shape[1]), jnp.float32)], axis=1
        )[None]

    @pl.when(i < nsteps - 1)
    def _score_step():
        adj = adj_ref[0]
        out = jnp.dot(adj, xf_ref[0], preferred_element_type=jnp.float32)
        s = jnp.dot(out, w_ref[...], preferred_element_type=jnp.float32)
        sig = 1.0 / (1.0 + jnp.exp(-s))
        sig_acc[pl.ds(i * rows, rows), :] = sig.reshape(rows, 128)

    @pl.when(i == nsteps - 1)
    def _sort_step():
        M = N // 128
        for b in range(4):
            top_ref[pl.ds(b, 1)] = _sort_tile(sig_acc[pl.ds(b * M, M), :], b, N)


def _fused(adj, xf, W):
    B, N, K = xf.shape
    last = B * _NB - 1
    return pl.pallas_call(
        _fused_body,
        grid=(B * _NB + 1,),
        in_specs=[
            pl.BlockSpec(
                (1, _BN, N),
                lambda i: (jnp.minimum(i, last) // _NB, jnp.minimum(i, last) % _NB, 0),
            ),
            pl.BlockSpec((1, N, K), lambda i: (jnp.minimum(i // _NB, B - 1), 0, 0)),
            pl.BlockSpec((K, 1), lambda i: (0, 0)),
        ],
        out_specs=[
            pl.BlockSpec((B, 1, N // 2), lambda i: (0, 0, 0)),
            pl.BlockSpec((1, N, 128), lambda i: (jnp.minimum(i // _NB, B - 1), 0, 0)),
        ],
        out_shape=[
            jax.ShapeDtypeStruct((B, 1, N // 2), jnp.int32),
            jax.ShapeDtypeStruct((B, N, 128), jnp.float32),
        ],
        scratch_shapes=[pltpu.VMEM((B * N // 128, 128), jnp.float32)],
    )(adj, xf, W)


def _gather_body(xf_hbm, idx_hbm, out_hbm, idx_v, rows_v, sem):
    wid = lax.axis_index("s") * 2 + lax.axis_index("c")
    pltpu.sync_copy(idx_hbm.at[pl.ds(wid * 2, 2)], idx_v)
    cps = [
        pltpu.async_copy(xf_hbm.at[idx_v.at[c]], rows_v.at[c], sem) for c in range(2)
    ]
    for cp in cps:
        cp.wait()
    for c in range(2):
        pltpu.sync_copy(rows_v.at[c], out_hbm.at[pl.ds(wid * 256 + c * 128, 128)])


def _gather(xf_flat, idx_2d):
    R, K = xf_flat.shape
    nrows = idx_2d.shape[0] * idx_2d.shape[1]
    return pl.kernel(
        _gather_body,
        out_type=jax.ShapeDtypeStruct((nrows, K), jnp.float32),
        mesh=plsc.VectorSubcoreMesh(core_axis_name="c", subcore_axis_name="s"),
        scratch_types=[
            pltpu.VMEM((2, 128), jnp.int32),
            pltpu.VMEM((2, 128, K), jnp.float32),
            pltpu.SemaphoreType.DMA,
        ],
    )(xf_flat, idx_2d)


@jax.jit
def kernel(x, adj, W):
    B, F, N, T = x.shape
    K = T * F
    xf = jnp.transpose(x, (0, 2, 3, 1)).reshape(B, N, K)
    top, xf_pad = _fused(adj, xf, W)
    g = _gather(xf_pad.reshape(B * N, 128), top.reshape(-1, 128))[:, :K]
    return g.reshape(B, N // 2, T, F).transpose(0, 3, 1, 2)


# speedup vs baseline: 1.3166x; 1.0060x over previous
import jax
import jax.numpy as jnp
from jax import lax
from jax.experimental import pallas as pl
from jax.experimental.pallas import tpu as pltpu
from jax.experimental.pallas import tpu_sc as plsc

_BN = 1024
_NB = 4


def _sort_tile(sig_tile, b, N):
    L = 128
    M = N // L
    key = lax.bitcast_convert_type(sig_tile, jnp.int32)
    lane = lax.broadcasted_iota(jnp.int32, (M, L), 1)
    row = lax.broadcasted_iota(jnp.int32, (M, L), 0)
    idx = row * L + lane
    k = 2
    while k <= N:
        j = k // 2
        while j >= 1:
            if j < L:
                lower = (lane & j) == 0
                ax, d = 1, j
            else:
                d = j // L
                lower = (row & d) == 0
                ax = 0
            kp = jnp.where(lower, jnp.roll(key, -d, axis=ax), jnp.roll(key, d, axis=ax))
            ip = jnp.where(lower, jnp.roll(idx, -d, axis=ax), jnp.roll(idx, d, axis=ax))
            asc = ((lane & k) == 0) if k < L else ((row & (k // L)) == 0)
            klo = jnp.where(lower, key, kp)
            khi = jnp.where(lower, kp, key)
            ilo = jnp.where(lower, idx, ip)
            ihi = jnp.where(lower, ip, idx)
            good = (klo > khi) | ((klo == khi) & (ilo < ihi))
            swap = good ^ asc
            key = jnp.where(swap, kp, key)
            idx = jnp.where(swap, ip, idx)
            j //= 2
        k *= 2
    return idx[: M // 2, :] + b * N


def _fused_body(adj_ref, xf_ref, w_ref, top_ref, xfp_ref, sig_acc):
    i = pl.program_id(0)
    nsteps = pl.num_programs(0)
    N = xf_ref.shape[1]
    rows = _BN // 128

    @pl.when((i % _NB == 0) & (i < nsteps - 1))
    def _emit_table():
        xf = xf_ref[0]
        xfp_ref[...] = jnp.concatenate(
            [xf, jnp.zeros((N, 128 - xf.shape[1]), jnp.float32)], axis=1
        )[None]

    @pl.when(i < nsteps - 1)
    def _score_step():
        adj = adj_ref[0]
        out = jnp.dot(adj, xf_ref[0], preferred_element_type=jnp.float32)
        s = jnp.dot(out, w_ref[...], preferred_element_type=jnp.float32)
        sig = 1.0 / (1.0 + jnp.exp(-s))
        sig_acc[pl.ds(i * rows, rows), :] = sig.reshape(rows, 128)

    @pl.when(i == nsteps - 1)
    def _sort_step():
        M = N // 128
        for b in range(4):
            top_ref[pl.ds(b * (M // 2), M // 2), :] = _sort_tile(
                sig_acc[pl.ds(b * M, M), :], b, N
            )


def _fused(adj, xf, W):
    B, N, K = xf.shape
    last = B * _NB - 1
    return pl.pallas_call(
        _fused_body,
        grid=(B * _NB + 1,),
        in_specs=[
            pl.BlockSpec(
                (1, _BN, N),
                lambda i: (jnp.minimum(i, last) // _NB, jnp.minimum(i, last) % _NB, 0),
            ),
            pl.BlockSpec((1, N, K), lambda i: (jnp.minimum(i // _NB, B - 1), 0, 0)),
            pl.BlockSpec((K, 1), lambda i: (0, 0)),
        ],
        out_specs=[
            pl.BlockSpec((B * N // 256, 128), lambda i: (0, 0)),
            pl.BlockSpec((1, N, 128), lambda i: (jnp.minimum(i // _NB, B - 1), 0, 0)),
        ],
        out_shape=[
            jax.ShapeDtypeStruct((B * N // 256, 128), jnp.int32),
            jax.ShapeDtypeStruct((B, N, 128), jnp.float32),
        ],
        scratch_shapes=[pltpu.VMEM((B * N // 128, 128), jnp.float32)],
    )(adj, xf, W)


def _gather_body(xf_hbm, idx_hbm, out_hbm, idx_v, rows_v, sem):
    wid = lax.axis_index("s") * 2 + lax.axis_index("c")
    pltpu.sync_copy(idx_hbm.at[pl.ds(wid * 2, 2)], idx_v)
    cps = [
        pltpu.async_copy(xf_hbm.at[idx_v.at[c]], rows_v.at[c], sem) for c in range(2)
    ]
    for cp in cps:
        cp.wait()
    for c in range(2):
        pltpu.sync_copy(rows_v.at[c], out_hbm.at[pl.ds(wid * 256 + c * 128, 128)])


def _gather(xf_flat, idx_2d):
    R, K = xf_flat.shape
    nrows = idx_2d.shape[0] * idx_2d.shape[1]
    return pl.kernel(
        _gather_body,
        out_type=jax.ShapeDtypeStruct((nrows, 128), jnp.float32),
        mesh=plsc.VectorSubcoreMesh(core_axis_name="c", subcore_axis_name="s"),
        scratch_types=[
            pltpu.VMEM((2, 128), jnp.int32),
            pltpu.VMEM((2, 128, K), jnp.float32),
            pltpu.SemaphoreType.DMA,
        ],
    )(xf_flat, idx_2d)


@jax.jit
def kernel(x, adj, W):
    B, F, N, T = x.shape
    K = T * F
    xf = jnp.transpose(x, (0, 2, 3, 1)).reshape(B, N, K)
    top, xf_pad = _fused(adj, xf, W)
    g = _gather(xf_pad.reshape(B * N, 128), top)[:, :K]
    return g.reshape(B, N // 2, T, F).transpose(0, 3, 1, 2)
